# baseline (speedup 1.0000x reference)
import functools

import jax
import jax.numpy as jnp
from jax import lax
from jax.experimental import pallas as pl
from jax.experimental.pallas import tpu as pltpu
from jax.experimental.pallas import tpu_sc as plsc

VOCAB = 1000000
EMBED_DIM = 64
D_MODEL = 128
B = 1024
L = 200
PAD_ID = 0

N_TOK = B * L
N_SC = 2
N_TEC = 16
N_W = N_SC * N_TEC
TOK_PER_W = N_TOK // N_W
CHUNK = 128
N_CHUNK = TOK_PER_W // CHUNK

_sc_mesh = plsc.VectorSubcoreMesh(core_axis_name="c", subcore_axis_name="s")


@functools.partial(
    pl.kernel,
    mesh=_sc_mesh,
    out_type=jax.ShapeDtypeStruct((N_TOK, D_MODEL), jnp.float32),
    scratch_types=[
        pltpu.VMEM((2, CHUNK), jnp.int32),
        pltpu.VMEM((2, CHUNK, D_MODEL), jnp.float32),
        pltpu.SemaphoreType.DMA,
        pltpu.SemaphoreType.DMA,
        pltpu.SemaphoreType.DMA,
    ],
)
def _sc_gather(g_hbm, idx_hbm, out_hbm, idxc_v, rows_v, gsem, osem0, osem1):
    wid = lax.axis_index("s") * N_SC + lax.axis_index("c")
    base0 = wid * TOK_PER_W

    def body(i, carry):
        for p, osem in ((0, osem0), (1, osem1)):
            j = 2 * i + p
            base = base0 + j * CHUNK

            @pl.when(i >= 1)
            def _():
                pltpu.make_async_copy(
                    out_hbm.at[pl.ds(0, CHUNK)], rows_v.at[p], osem
                ).wait()

            pltpu.sync_copy(idx_hbm.at[pl.ds(base, CHUNK)], idxc_v.at[p])
            pltpu.async_copy(g_hbm.at[idxc_v.at[p]], rows_v.at[p], gsem).wait()
            pltpu.async_copy(rows_v.at[p], out_hbm.at[pl.ds(base, CHUNK)], osem)
        return carry

    lax.fori_loop(0, N_CHUNK // 2, body, 0)

    pltpu.make_async_copy(out_hbm.at[pl.ds(0, CHUNK)], rows_v.at[0], osem0).wait()
    pltpu.make_async_copy(out_hbm.at[pl.ds(0, CHUNK)], rows_v.at[1], osem1).wait()


VBLK = 16384
G_GRID = -(-VOCAB // VBLK)


def _g_body(tT_ref, w_ref, b_ref, g_ref):
    y = lax.dot_general(
        tT_ref[...], w_ref[...],
        dimension_numbers=(((0,), (0,)), ((), ())),
        preferred_element_type=jnp.float32,
    )
    y = y + b_ref[...]
    g_ref[...] = y * 0.5 * (1.0 + lax.erf(y * 0.7071067811865476))


_g_proj = pl.pallas_call(
    _g_body,
    grid=(G_GRID,),
    in_specs=[
        pl.BlockSpec((EMBED_DIM, VBLK), lambda i: (0, i)),
        pl.BlockSpec((EMBED_DIM, D_MODEL), lambda i: (0, 0)),
        pl.BlockSpec((1, D_MODEL), lambda i: (0, 0)),
    ],
    out_specs=pl.BlockSpec((VBLK, D_MODEL), lambda i: (i, 0)),
    out_shape=jax.ShapeDtypeStruct((VOCAB, D_MODEL), jnp.float32),
    compiler_params=pltpu.CompilerParams(dimension_semantics=("parallel",)),
)


def _mask_body(idx_ref, mask_ref):
    mask_ref[...] = idx_ref[...] == PAD_ID


_mask_call = pl.pallas_call(
    _mask_body,
    grid=(1,),
    in_specs=[pl.BlockSpec((N_TOK // 128, 128), lambda i: (0, 0))],
    out_specs=pl.BlockSpec((N_TOK // 128, 128), lambda i: (0, 0)),
    out_shape=jax.ShapeDtypeStruct((N_TOK // 128, 128), jnp.bool_),
)


def kernel(indices, table, W, b):
    idx_flat = indices.reshape(-1).astype(jnp.int32)
    g = _g_proj(table.T, W, b.reshape(1, D_MODEL))
    out = _sc_gather(g, idx_flat)
    mask = _mask_call(idx_flat.reshape(N_TOK // 128, 128))
    return out.reshape(B, L, D_MODEL), mask.reshape(B, L)


# speedup vs baseline: 1.0200x; 1.0200x over previous
import functools

import jax
import jax.numpy as jnp
from jax import lax
from jax.experimental import pallas as pl
from jax.experimental.pallas import tpu as pltpu
from jax.experimental.pallas import tpu_sc as plsc

VOCAB = 1000000
EMBED_DIM = 64
D_MODEL = 128
B = 1024
L = 200
PAD_ID = 0

N_TOK = B * L
N_SC = 2
N_TEC = 16
N_W = N_SC * N_TEC
TOK_PER_W = N_TOK // N_W
CHUNK = 128
N_CHUNK = TOK_PER_W // CHUNK

_sc_mesh = plsc.VectorSubcoreMesh(core_axis_name="c", subcore_axis_name="s")


@functools.partial(
    pl.kernel,
    mesh=_sc_mesh,
    out_type=jax.ShapeDtypeStruct((N_TOK, D_MODEL), jnp.float32),
    scratch_types=[
        pltpu.VMEM((2, CHUNK), jnp.int32),
        pltpu.VMEM((2, CHUNK, D_MODEL), jnp.float32),
        pltpu.SemaphoreType.DMA,
        pltpu.SemaphoreType.DMA,
        pltpu.SemaphoreType.DMA,
    ],
)
def _sc_gather(g_hbm, idx_hbm, out_hbm, idxc_v, rows_v, gsem, osem0, osem1):
    wid = lax.axis_index("s") * N_SC + lax.axis_index("c")
    base0 = wid * TOK_PER_W

    def body(i, carry):
        for p, osem in ((0, osem0), (1, osem1)):
            j = 2 * i + p
            base = base0 + j * CHUNK

            @pl.when(i >= 1)
            def _():
                pltpu.make_async_copy(
                    out_hbm.at[pl.ds(0, CHUNK)], rows_v.at[p], osem
                ).wait()

            pltpu.sync_copy(idx_hbm.at[pl.ds(base, CHUNK)], idxc_v.at[p])
            pltpu.async_copy(g_hbm.at[idxc_v.at[p]], rows_v.at[p], gsem).wait()
            pltpu.async_copy(rows_v.at[p], out_hbm.at[pl.ds(base, CHUNK)], osem)
        return carry

    lax.fori_loop(0, N_CHUNK // 2, body, 0)

    pltpu.make_async_copy(out_hbm.at[pl.ds(0, CHUNK)], rows_v.at[0], osem0).wait()
    pltpu.make_async_copy(out_hbm.at[pl.ds(0, CHUNK)], rows_v.at[1], osem1).wait()


VBLK = 32768
G_GRID = -(-VOCAB // VBLK)


def _g_body(tT_ref, w_ref, b_ref, g_ref):
    y = lax.dot_general(
        tT_ref[...], w_ref[...],
        dimension_numbers=(((0,), (0,)), ((), ())),
        preferred_element_type=jnp.float32,
    )
    y = y + b_ref[...]
    g_ref[...] = y * 0.5 * (1.0 + lax.erf(y * 0.7071067811865476))


_g_proj = pl.pallas_call(
    _g_body,
    grid=(G_GRID,),
    in_specs=[
        pl.BlockSpec((EMBED_DIM, VBLK), lambda i: (0, i)),
        pl.BlockSpec((EMBED_DIM, D_MODEL), lambda i: (0, 0)),
        pl.BlockSpec((1, D_MODEL), lambda i: (0, 0)),
    ],
    out_specs=pl.BlockSpec((VBLK, D_MODEL), lambda i: (i, 0)),
    out_shape=jax.ShapeDtypeStruct((VOCAB, D_MODEL), jnp.float32),
)


def _mask_body(idx_ref, mask_ref):
    mask_ref[...] = idx_ref[...] == PAD_ID


_mask_call = pl.pallas_call(
    _mask_body,
    grid=(1,),
    in_specs=[pl.BlockSpec((N_TOK // 128, 128), lambda i: (0, 0))],
    out_specs=pl.BlockSpec((N_TOK // 128, 128), lambda i: (0, 0)),
    out_shape=jax.ShapeDtypeStruct((N_TOK // 128, 128), jnp.bool_),
)


def kernel(indices, table, W, b):
    idx_flat = indices.reshape(-1).astype(jnp.int32)
    g = _g_proj(table.T, W, b.reshape(1, D_MODEL))
    out = _sc_gather(g, idx_flat)
    mask = _mask_call(idx_flat.reshape(N_TOK // 128, 128))
    return out.reshape(B, L, D_MODEL), mask.reshape(B, L)


# speedup vs baseline: 1.0948x; 1.0734x over previous
import functools

import jax
import jax.numpy as jnp
from jax import lax
from jax.experimental import pallas as pl
from jax.experimental.pallas import tpu as pltpu
from jax.experimental.pallas import tpu_sc as plsc

VOCAB = 1000000
EMBED_DIM = 64
D_MODEL = 128
B = 1024
L = 200
PAD_ID = 0

N_TOK = B * L
N_SC = 2
N_TEC = 16
N_W = N_SC * N_TEC
TOK_PER_W = N_TOK // N_W
CHUNK = 128
N_CHUNK = TOK_PER_W // CHUNK

_sc_mesh = plsc.VectorSubcoreMesh(core_axis_name="c", subcore_axis_name="s")


@functools.partial(
    pl.kernel,
    mesh=_sc_mesh,
    out_type=jax.ShapeDtypeStruct((N_TOK, D_MODEL), jnp.float32),
    scratch_types=[
        pltpu.VMEM((4, CHUNK), jnp.int32),
        pltpu.VMEM((4, CHUNK, D_MODEL), jnp.float32),
        pltpu.SemaphoreType.DMA,
        pltpu.SemaphoreType.DMA,
        pltpu.SemaphoreType.DMA,
        pltpu.SemaphoreType.DMA,
        pltpu.SemaphoreType.DMA,
        pltpu.SemaphoreType.DMA,
        pltpu.SemaphoreType.DMA,
        pltpu.SemaphoreType.DMA,
    ],
)
def _sc_gather(g_hbm, idx_hbm, out_hbm, idxc_v, rows_v,
               gsem0, gsem1, gsem2, gsem3, osem0, osem1, osem2, osem3):
    wid = lax.axis_index("s") * N_SC + lax.axis_index("c")
    base0 = wid * TOK_PER_W
    gsems = (gsem0, gsem1, gsem2, gsem3)
    osems = (osem0, osem1, osem2, osem3)

    def launch(j, p, wait_osem):
        if wait_osem is not None:
            pltpu.make_async_copy(
                out_hbm.at[pl.ds(0, CHUNK)], rows_v.at[p], wait_osem
            ).wait()
        pltpu.sync_copy(idx_hbm.at[pl.ds(base0 + j * CHUNK, CHUNK)],
                        idxc_v.at[p])
        pltpu.async_copy(g_hbm.at[idxc_v.at[p]], rows_v.at[p], gsems[p])

    def retire(j, p):
        pltpu.make_async_copy(
            out_hbm.at[pl.ds(0, CHUNK)], rows_v.at[p], gsems[p]
        ).wait()
        pltpu.async_copy(
            rows_v.at[p], out_hbm.at[pl.ds(base0 + j * CHUNK, CHUNK)],
            osems[p],
        )

    launch(0, 0, None)
    launch(1, 1, None)

    def body(i, carry):
        j = 4 * i
        retire(j, 0)

        @pl.when(i >= 1)
        def _():
            pltpu.make_async_copy(
                out_hbm.at[pl.ds(0, CHUNK)], rows_v.at[2], osem2
            ).wait()

        pltpu.sync_copy(idx_hbm.at[pl.ds(base0 + (j + 2) * CHUNK, CHUNK)],
                        idxc_v.at[2])
        pltpu.async_copy(g_hbm.at[idxc_v.at[2]], rows_v.at[2], gsem2)

        retire(j + 1, 1)

        @pl.when(i >= 1)
        def _():
            pltpu.make_async_copy(
                out_hbm.at[pl.ds(0, CHUNK)], rows_v.at[3], osem3
            ).wait()

        pltpu.sync_copy(idx_hbm.at[pl.ds(base0 + (j + 3) * CHUNK, CHUNK)],
                        idxc_v.at[3])
        pltpu.async_copy(g_hbm.at[idxc_v.at[3]], rows_v.at[3], gsem3)

        retire(j + 2, 2)
        pltpu.make_async_copy(
            out_hbm.at[pl.ds(0, CHUNK)], rows_v.at[0], osem0
        ).wait()
        pltpu.sync_copy(idx_hbm.at[pl.ds(base0 + (j + 4) * CHUNK, CHUNK)],
                        idxc_v.at[0])
        pltpu.async_copy(g_hbm.at[idxc_v.at[0]], rows_v.at[0], gsem0)

        retire(j + 3, 3)
        pltpu.make_async_copy(
            out_hbm.at[pl.ds(0, CHUNK)], rows_v.at[1], osem1
        ).wait()
        pltpu.sync_copy(idx_hbm.at[pl.ds(base0 + (j + 5) * CHUNK, CHUNK)],
                        idxc_v.at[1])
        pltpu.async_copy(g_hbm.at[idxc_v.at[1]], rows_v.at[1], gsem1)
        return carry

    lax.fori_loop(0, (N_CHUNK - 2) // 4, body, 0)

    retire(N_CHUNK - 2, 0)
    retire(N_CHUNK - 1, 1)

    pltpu.make_async_copy(out_hbm.at[pl.ds(0, CHUNK)], rows_v.at[2], osem2).wait()
    pltpu.make_async_copy(out_hbm.at[pl.ds(0, CHUNK)], rows_v.at[3], osem3).wait()
    pltpu.make_async_copy(out_hbm.at[pl.ds(0, CHUNK)], rows_v.at[0], osem0).wait()
    pltpu.make_async_copy(out_hbm.at[pl.ds(0, CHUNK)], rows_v.at[1], osem1).wait()


VBLK = 32768
G_GRID = -(-VOCAB // VBLK)


def _g_body(tT_ref, w_ref, b_ref, g_ref):
    y = lax.dot_general(
        tT_ref[...], w_ref[...],
        dimension_numbers=(((0,), (0,)), ((), ())),
        preferred_element_type=jnp.float32,
    )
    y = y + b_ref[...]
    g_ref[...] = y * 0.5 * (1.0 + lax.erf(y * 0.7071067811865476))


_g_proj = pl.pallas_call(
    _g_body,
    grid=(G_GRID,),
    in_specs=[
        pl.BlockSpec((EMBED_DIM, VBLK), lambda i: (0, i)),
        pl.BlockSpec((EMBED_DIM, D_MODEL), lambda i: (0, 0)),
        pl.BlockSpec((1, D_MODEL), lambda i: (0, 0)),
    ],
    out_specs=pl.BlockSpec((VBLK, D_MODEL), lambda i: (i, 0)),
    out_shape=jax.ShapeDtypeStruct((VOCAB, D_MODEL), jnp.float32),
)


def _mask_body(idx_ref, mask_ref):
    mask_ref[...] = idx_ref[...] == PAD_ID


_mask_call = pl.pallas_call(
    _mask_body,
    grid=(1,),
    in_specs=[pl.BlockSpec((N_TOK // 128, 128), lambda i: (0, 0))],
    out_specs=pl.BlockSpec((N_TOK // 128, 128), lambda i: (0, 0)),
    out_shape=jax.ShapeDtypeStruct((N_TOK // 128, 128), jnp.bool_),
)


def kernel(indices, table, W, b):
    idx_flat = indices.reshape(-1).astype(jnp.int32)
    g = _g_proj(table.T, W, b.reshape(1, D_MODEL))
    out = _sc_gather(g, idx_flat)
    mask = _mask_call(idx_flat.reshape(N_TOK // 128, 128))
    return out.reshape(B, L, D_MODEL), mask.reshape(B, L)
